# baseline (speedup 1.0000x reference)
import functools

import jax
import jax.numpy as jnp
from jax import lax
from jax.experimental import pallas as pl
from jax.experimental.pallas import tpu as pltpu
from jax.experimental.pallas import tpu_sc as plsc

N_NODES = 10000
N_EDGES = 320000
DIM = 128
N_GRAPHS = 200
N_LAYERS = 3
N_STATS = 7
N_MOTIF = 50

NPAD = 10112
DUMMY_ROW = 10000
N_DUMMY = NPAD - DUMMY_ROW
CHUNK = 128
EPAD = 327680
CPW = EPAD // (16 * CHUNK)
PH1 = 128
PH2 = CPW - PH1
ROWS_PER_TILE = NPAD // 16

BPAD = 256
ROW_BLOCK = NPAD // 8
N_BLOCKS = 8
ADJ_COLS = N_MOTIF * N_MOTIF
ADJ_PAD = 3072
ADJ_BLOCK = ADJ_PAD // N_BLOCKS
HID = DIM // 2


def _sc_agg_body(h_hbm, zeros_hbm, src_hbm, dst_hbm, out,
                 src_v, ring, rows0, rows1, acc_sh,
                 sg0, sg1, ss0, ss1, sr0, sr1):
    sid = lax.axis_index("s")
    row0 = sid * ROWS_PER_TILE
    rows = (rows0, rows1)
    sg = (sg0, sg1)
    ss = (ss0, ss1)
    sr = (sr0, sr1)

    pltpu.sync_copy(zeros_hbm.at[pl.ds(row0, ROWS_PER_TILE)],
                    acc_sh.at[pl.ds(row0, ROWS_PER_TILE)])

    def src_ix(j):
        return src_v.at[pl.ds(pl.multiple_of(j * CHUNK, 8), CHUNK)]

    def run_phase(n, j_off):
        pltpu.async_copy(dst_hbm.at[sid, j_off], ring.at[0], sr0)
        pltpu.async_copy(h_hbm.at[src_ix(0)], rows0, sg0)

        @pl.loop(0, n, step=2)
        def _(j0):
            for b in range(2):
                j = j0 + b
                bg = 1 - b

                @pl.when(j >= 1)
                def _():
                    pltpu.make_async_copy(
                        rows[bg], acc_sh.at[ring.at[bg]], ss[bg]).wait()

                @pl.when(j + 1 < n)
                def _():
                    pltpu.async_copy(dst_hbm.at[sid, j_off + j + 1],
                                     ring.at[bg], sr[bg])
                    pltpu.async_copy(h_hbm.at[src_ix(j + 1)], rows[bg],
                                     sg[bg])

                pltpu.make_async_copy(h_hbm.at[src_ix(j)], rows[b],
                                      sg[b]).wait()
                pltpu.make_async_copy(dst_hbm.at[sid, j_off + j],
                                      ring.at[b], sr[b]).wait()
                pltpu.async_copy(rows[b], acc_sh.at[ring.at[b]], ss[b],
                                 add=True)

        pltpu.make_async_copy(rows1, acc_sh.at[ring.at[1]], ss1).wait()

    pltpu.sync_copy(src_hbm.at[sid, pl.ds(0, PH1 * CHUNK)],
                    src_v.at[pl.ds(0, PH1 * CHUNK)])
    plsc.subcore_barrier()
    run_phase(PH1, 0)

    pltpu.sync_copy(src_hbm.at[sid, pl.ds(PH1 * CHUNK, PH2 * CHUNK)],
                    src_v.at[pl.ds(0, PH2 * CHUNK)])
    run_phase(PH2, PH1)

    plsc.subcore_barrier()
    pltpu.sync_copy(acc_sh.at[pl.ds(row0, ROWS_PER_TILE)],
                    out.at[pl.ds(row0, ROWS_PER_TILE)])


@functools.lru_cache(maxsize=1)
def _make_sc_agg():
    return pl.kernel(
        _sc_agg_body,
        out_type=jax.ShapeDtypeStruct((NPAD, DIM), jnp.float32),
        mesh=plsc.VectorSubcoreMesh(core_axis_name="c",
                                    subcore_axis_name="s", num_cores=1),
        scratch_types=[
            pltpu.VMEM((PH1 * CHUNK,), jnp.int32),
            pltpu.VMEM((2, CHUNK), jnp.int32),
            pltpu.VMEM((CHUNK, DIM), jnp.float32),
            pltpu.VMEM((CHUNK, DIM), jnp.float32),
            pltpu.VMEM_SHARED((NPAD, DIM), jnp.float32),
            pltpu.SemaphoreType.DMA,
            pltpu.SemaphoreType.DMA,
            pltpu.SemaphoreType.DMA,
            pltpu.SemaphoreType.DMA,
            pltpu.SemaphoreType.DMA,
            pltpu.SemaphoreType.DMA,
        ],
    )


def _sc_agg(h, zeros_h, src2, dst3):
    return _make_sc_agg()(h, zeros_h, src2, dst3)


def _dot(a, b):
    return lax.dot_general(a, b, (((1,), (0,)), ((), ())),
                           precision=lax.Precision.HIGHEST,
                           preferred_element_type=jnp.float32)


def _mlp_block(z, w1t, b1, a1, scale, beta, w2t, b2, a2):
    z = _dot(z, w1t) + b1
    z = jnp.where(z >= 0, z, a1 * z)
    z = z * scale + beta
    z = _dot(z, w2t) + b2
    z = jnp.where(z >= 0, z, a2 * z)
    return jnp.maximum(z, 0.0)


def _mlp_kernel(h_ref, p_ref, w1t_ref, b1_ref, a1_ref, scale_ref, beta_ref,
                w2t_ref, b2_ref, a2_ref, out_ref):
    z = h_ref[...] + p_ref[...]
    out_ref[...] = _mlp_block(
        z, w1t_ref[...], b1_ref[...], a1_ref[0, 0],
        scale_ref[...], beta_ref[...], w2t_ref[...], b2_ref[...],
        a2_ref[0, 0])


def _mlp_call(h, p, w1t, b1, a1, scale, beta, w2t, b2, a2):
    blk = lambda i: (i, 0)
    fixed = lambda i: (0, 0)
    return pl.pallas_call(
        _mlp_kernel,
        grid=(N_BLOCKS,),
        in_specs=[
            pl.BlockSpec((ROW_BLOCK, DIM), blk),
            pl.BlockSpec((ROW_BLOCK, DIM), blk),
            pl.BlockSpec((DIM, DIM), fixed),
            pl.BlockSpec((1, DIM), fixed),
            pl.BlockSpec((1, 1), fixed),
            pl.BlockSpec((1, DIM), fixed),
            pl.BlockSpec((1, DIM), fixed),
            pl.BlockSpec((DIM, DIM), fixed),
            pl.BlockSpec((1, DIM), fixed),
            pl.BlockSpec((1, 1), fixed),
        ],
        out_specs=pl.BlockSpec((ROW_BLOCK, DIM), blk),
        out_shape=jax.ShapeDtypeStruct((NPAD, DIM), jnp.float32),
    )(h, p, w1t, b1, a1, scale, beta, w2t, b2, a2)


def _final_kernel(h_ref, p_ref, w1t_ref, b1_ref, a1_ref, scale_ref,
                  beta_ref, w2t_ref, b2_ref, a2_ref, batch_ref, adj_ref,
                  fc1adjt_ref, stats_ref, fc1poolt_ref, fc1statst_ref,
                  fc1b_ref, fc2t_ref, fc2b_ref, xl_ref, out_ref,
                  pool_acc, adj_acc):
    i = pl.program_id(0)
    z = h_ref[...] + p_ref[...]
    h = _mlp_block(
        z, w1t_ref[...], b1_ref[...], a1_ref[0, 0],
        scale_ref[...], beta_ref[...], w2t_ref[...], b2_ref[...],
        a2_ref[0, 0])

    b_ids = batch_ref[0, 0, :]
    seg = lax.broadcasted_iota(jnp.int32, (BPAD, ROW_BLOCK), 0)
    onehot_t = (seg == jnp.reshape(b_ids, (1, ROW_BLOCK))).astype(jnp.float32)
    pool_contrib = _dot(onehot_t, h)
    adj_contrib = _dot(adj_ref[...], fc1adjt_ref[...])

    @pl.when(i == 0)
    def _():
        pool_acc[...] = pool_contrib
        adj_acc[...] = adj_contrib

    @pl.when(i != 0)
    def _():
        pool_acc[...] += pool_contrib
        adj_acc[...] += adj_contrib

    @pl.when(i == N_BLOCKS - 1)
    def _():
        xt = (_dot(pool_acc[...], fc1poolt_ref[...])
              + _dot(stats_ref[...], fc1statst_ref[...])
              + adj_acc[...] + fc1b_ref[...])
        xt = jnp.maximum(xt, 0.0)
        xl_ref[...] = xt
        logits = _dot(xt, fc2t_ref[...]) + fc2b_ref[...]
        out_ref[...] = jax.nn.sigmoid(logits)


def _final_call(h, p, w1t, b1, a1, scale, beta, w2t, b2, a2, batch3d,
                adj_pad, fc1adjt, stats_pad, fc1poolt, fc1statst, fc1b,
                fc2t, fc2b):
    fixed = lambda i: (0, 0)
    return pl.pallas_call(
        _final_kernel,
        grid=(N_BLOCKS,),
        in_specs=[
            pl.BlockSpec((ROW_BLOCK, DIM), lambda i: (i, 0)),
            pl.BlockSpec((ROW_BLOCK, DIM), lambda i: (i, 0)),
            pl.BlockSpec((DIM, DIM), fixed),
            pl.BlockSpec((1, DIM), fixed),
            pl.BlockSpec((1, 1), fixed),
            pl.BlockSpec((1, DIM), fixed),
            pl.BlockSpec((1, DIM), fixed),
            pl.BlockSpec((DIM, DIM), fixed),
            pl.BlockSpec((1, DIM), fixed),
            pl.BlockSpec((1, 1), fixed),
            pl.BlockSpec((1, 1, ROW_BLOCK), lambda i: (i, 0, 0)),
            pl.BlockSpec((BPAD, ADJ_BLOCK), lambda i: (0, i)),
            pl.BlockSpec((ADJ_BLOCK, DIM), lambda i: (i, 0)),
            pl.BlockSpec((BPAD, DIM), fixed),
            pl.BlockSpec((DIM, DIM), fixed),
            pl.BlockSpec((DIM, DIM), fixed),
            pl.BlockSpec((1, DIM), fixed),
            pl.BlockSpec((DIM, DIM), fixed),
            pl.BlockSpec((1, DIM), fixed),
        ],
        out_specs=[
            pl.BlockSpec((BPAD, DIM), fixed),
            pl.BlockSpec((BPAD, DIM), fixed),
        ],
        out_shape=[
            jax.ShapeDtypeStruct((BPAD, DIM), jnp.float32),
            jax.ShapeDtypeStruct((BPAD, DIM), jnp.float32),
        ],
        scratch_shapes=[
            pltpu.VMEM((BPAD, DIM), jnp.float32),
            pltpu.VMEM((BPAD, DIM), jnp.float32),
        ],
    )(h, p, w1t, b1, a1, scale, beta, w2t, b2, a2, batch3d, adj_pad,
      fc1adjt, stats_pad, fc1poolt, fc1statst, fc1b, fc2t, fc2b)


def kernel(x, edge_index, batch, stats, adj, W1, b1, a1, bn_gamma, bn_beta,
           W2, b2, a2, fc1_w, fc1_b, fc2_w, fc2_b):
    f32 = jnp.float32

    src = edge_index[0].astype(jnp.int32)
    dst = edge_index[1].astype(jnp.int32)
    pad_e = EPAD - N_EDGES
    src_p = jnp.concatenate([src, jnp.zeros((pad_e,), jnp.int32)])
    dummy_dst = DUMMY_ROW + (jnp.arange(pad_e, dtype=jnp.int32) % N_DUMMY)
    dst_p = jnp.concatenate([dst, dummy_dst])

    src2 = src_p.reshape(16, CPW * CHUNK)
    dst3 = dst_p.reshape(16, CPW, CHUNK)

    h = jnp.concatenate(
        [x.astype(f32), jnp.zeros((NPAD - N_NODES, DIM), f32)], axis=0)
    zeros_h = jnp.zeros((NPAD, DIM), f32)

    bn_scale = (bn_gamma * jax.lax.rsqrt(jnp.float32(1.0 + 1e-5))).astype(f32)

    batch_p = jnp.concatenate(
        [batch.astype(jnp.int32),
         jnp.full((NPAD - N_NODES,), BPAD - 1, jnp.int32)])
    batch3d = batch_p.reshape(N_BLOCKS, 1, ROW_BLOCK)

    adj_flat = adj.reshape(N_GRAPHS, ADJ_COLS).astype(f32)
    adj_pad = jnp.zeros((BPAD, ADJ_PAD), f32)
    adj_pad = adj_pad.at[:N_GRAPHS, :ADJ_COLS].set(adj_flat)

    stats_pad = jnp.zeros((BPAD, DIM), f32)
    stats_pad = stats_pad.at[:N_GRAPHS, :N_STATS].set(stats.astype(f32))

    fc1 = fc1_w.astype(f32)
    fc1poolt = jnp.zeros((DIM, DIM), f32)
    fc1poolt = fc1poolt.at[:, :HID].set(fc1[:, :DIM].T)
    fc1statst = jnp.zeros((DIM, DIM), f32)
    fc1statst = fc1statst.at[:N_STATS, :HID].set(fc1[:, DIM:DIM + N_STATS].T)
    fc1adjt = jnp.zeros((ADJ_PAD, DIM), f32)
    fc1adjt = fc1adjt.at[:ADJ_COLS, :HID].set(fc1[:, DIM + N_STATS:].T)
    fc1b = jnp.zeros((1, DIM), f32)
    fc1b = fc1b.at[0, :HID].set(fc1_b.astype(f32))
    fc2t = jnp.zeros((DIM, DIM), f32)
    fc2t = fc2t.at[:HID, 0].set(fc2_w.astype(f32)[0, :])
    fc2b = jnp.zeros((1, DIM), f32)
    fc2b = fc2b.at[0, 0].set(fc2_b.astype(f32)[0])

    for i in range(N_LAYERS):
        p = _sc_agg(h, zeros_h, src2, dst3)
        w1t = W1[i].T.astype(f32)
        w2t = W2[i].T.astype(f32)
        b1r = b1[i].reshape(1, DIM).astype(f32)
        b2r = b2[i].reshape(1, DIM).astype(f32)
        a1s = a1[i].reshape(1, 1).astype(f32)
        a2s = a2[i].reshape(1, 1).astype(f32)
        sc_r = bn_scale[i].reshape(1, DIM)
        be_r = bn_beta[i].reshape(1, DIM).astype(f32)
        if i < N_LAYERS - 1:
            h = _mlp_call(h, p, w1t, b1r, a1s, sc_r, be_r, w2t, b2r, a2s)
        else:
            xl, oo = _final_call(h, p, w1t, b1r, a1s, sc_r, be_r, w2t,
                                 b2r, a2s, batch3d, adj_pad, fc1adjt,
                                 stats_pad, fc1poolt, fc1statst, fc1b,
                                 fc2t, fc2b)

    out = oo[:N_GRAPHS, :1]
    x_l = xl[:N_GRAPHS, :HID]
    return (out, x_l)


# speedup vs baseline: 1.1453x; 1.1453x over previous
import functools

import jax
import jax.numpy as jnp
from jax import lax
from jax.experimental import pallas as pl
from jax.experimental.pallas import tpu as pltpu
from jax.experimental.pallas import tpu_sc as plsc

N_NODES = 10000
N_EDGES = 320000
DIM = 128
N_GRAPHS = 200
N_LAYERS = 3
N_STATS = 7
N_MOTIF = 50

NPAD = 10112
DUMMY_ROW = 10000
N_DUMMY = NPAD - DUMMY_ROW
CHUNK = 128
EPAD = 327680
N_CHUNKS = EPAD // CHUNK
PH1 = 128
PH2 = 22
CPW0 = PH1 + PH2
N1_SLOW = 8
N2_SLOW = 2
CPW1 = N1_SLOW + N2_SLOW
ROWS_PER_TILE = NPAD // 16

BPAD = 256
ROW_BLOCK = NPAD // 8
N_BLOCKS = 8
ADJ_COLS = N_MOTIF * N_MOTIF
ADJ_PAD = 3072
ADJ_BLOCK = ADJ_PAD // N_BLOCKS
HID = DIM // 2


def _sc_agg_body(h_hbm, zeros_hbm, src_hbm, dst_hbm, out,
                 src_v, ring, rows0, rows1, acc_sh,
                 sg0, sg1, ss0, ss1, sr0, sr1):
    cid = lax.axis_index("c")
    sid = lax.axis_index("s")
    wid = sid * 2 + cid
    row0 = sid * ROWS_PER_TILE
    rows = (rows0, rows1)
    sg = (sg0, sg1)
    ss = (ss0, ss1)
    sr = (sr0, sr1)

    pltpu.sync_copy(zeros_hbm.at[pl.ds(row0, ROWS_PER_TILE)],
                    acc_sh.at[pl.ds(row0, ROWS_PER_TILE)])

    def src_ix(j):
        return src_v.at[pl.ds(pl.multiple_of(j * CHUNK, 8), CHUNK)]

    def run_phase(n, j_off):
        pltpu.async_copy(dst_hbm.at[wid, j_off], ring.at[0], sr0)
        pltpu.async_copy(h_hbm.at[src_ix(0)], rows0, sg0)

        @pl.loop(0, n, step=2)
        def _(j0):
            for b in range(2):
                j = j0 + b
                bg = 1 - b

                @pl.when(j >= 1)
                def _():
                    pltpu.make_async_copy(
                        rows[bg], acc_sh.at[ring.at[bg]], ss[bg]).wait()

                @pl.when(j + 1 < n)
                def _():
                    pltpu.async_copy(dst_hbm.at[wid, j_off + j + 1],
                                     ring.at[bg], sr[bg])
                    pltpu.async_copy(h_hbm.at[src_ix(j + 1)], rows[bg],
                                     sg[bg])

                pltpu.make_async_copy(h_hbm.at[src_ix(j)], rows[b],
                                      sg[b]).wait()
                pltpu.make_async_copy(dst_hbm.at[wid, j_off + j],
                                      ring.at[b], sr[b]).wait()
                pltpu.async_copy(rows[b], acc_sh.at[ring.at[b]], ss[b],
                                 add=True)

        pltpu.make_async_copy(rows1, acc_sh.at[ring.at[1]], ss1).wait()

    n1 = jnp.where(cid == 0, PH1, N1_SLOW)
    n2 = jnp.where(cid == 0, PH2, N2_SLOW)

    pltpu.sync_copy(src_hbm.at[wid, pl.ds(0, PH1 * CHUNK)],
                    src_v.at[pl.ds(0, PH1 * CHUNK)])
    plsc.subcore_barrier()
    run_phase(n1, 0)

    pltpu.sync_copy(src_hbm.at[wid, pl.ds(PH1 * CHUNK, PH2 * CHUNK)],
                    src_v.at[pl.ds(0, PH2 * CHUNK)])
    run_phase(n2, PH1)

    plsc.subcore_barrier()
    pltpu.sync_copy(acc_sh.at[pl.ds(row0, ROWS_PER_TILE)],
                    out.at[cid, pl.ds(row0, ROWS_PER_TILE)])


@functools.lru_cache(maxsize=1)
def _make_sc_agg():
    return pl.kernel(
        _sc_agg_body,
        out_type=jax.ShapeDtypeStruct((2, NPAD, DIM), jnp.float32),
        mesh=plsc.VectorSubcoreMesh(core_axis_name="c",
                                    subcore_axis_name="s"),
        scratch_types=[
            pltpu.VMEM((PH1 * CHUNK,), jnp.int32),
            pltpu.VMEM((2, CHUNK), jnp.int32),
            pltpu.VMEM((CHUNK, DIM), jnp.float32),
            pltpu.VMEM((CHUNK, DIM), jnp.float32),
            pltpu.VMEM_SHARED((NPAD, DIM), jnp.float32),
            pltpu.SemaphoreType.DMA,
            pltpu.SemaphoreType.DMA,
            pltpu.SemaphoreType.DMA,
            pltpu.SemaphoreType.DMA,
            pltpu.SemaphoreType.DMA,
            pltpu.SemaphoreType.DMA,
        ],
    )


def _sc_agg(h, zeros_h, src2, dst3):
    return _make_sc_agg()(h, zeros_h, src2, dst3)


def _dot(a, b):
    return lax.dot_general(a, b, (((1,), (0,)), ((), ())),
                           precision=lax.Precision.HIGHEST,
                           preferred_element_type=jnp.float32)


def _mlp_block(z, w1t, b1, a1, scale, beta, w2t, b2, a2):
    z = _dot(z, w1t) + b1
    z = jnp.where(z >= 0, z, a1 * z)
    z = z * scale + beta
    z = _dot(z, w2t) + b2
    z = jnp.where(z >= 0, z, a2 * z)
    return jnp.maximum(z, 0.0)


def _mlp_kernel(h_ref, p_ref, w1t_ref, b1_ref, a1_ref, scale_ref, beta_ref,
                w2t_ref, b2_ref, a2_ref, out_ref):
    z = h_ref[...] + p_ref[0] + p_ref[1]
    out_ref[...] = _mlp_block(
        z, w1t_ref[...], b1_ref[...], a1_ref[0, 0],
        scale_ref[...], beta_ref[...], w2t_ref[...], b2_ref[...],
        a2_ref[0, 0])


def _mlp_call(h, p, w1t, b1, a1, scale, beta, w2t, b2, a2):
    blk = lambda i: (i, 0)
    fixed = lambda i: (0, 0)
    return pl.pallas_call(
        _mlp_kernel,
        grid=(N_BLOCKS,),
        in_specs=[
            pl.BlockSpec((ROW_BLOCK, DIM), blk),
            pl.BlockSpec((2, ROW_BLOCK, DIM), lambda i: (0, i, 0)),
            pl.BlockSpec((DIM, DIM), fixed),
            pl.BlockSpec((1, DIM), fixed),
            pl.BlockSpec((1, 1), fixed),
            pl.BlockSpec((1, DIM), fixed),
            pl.BlockSpec((1, DIM), fixed),
            pl.BlockSpec((DIM, DIM), fixed),
            pl.BlockSpec((1, DIM), fixed),
            pl.BlockSpec((1, 1), fixed),
        ],
        out_specs=pl.BlockSpec((ROW_BLOCK, DIM), blk),
        out_shape=jax.ShapeDtypeStruct((NPAD, DIM), jnp.float32),
    )(h, p, w1t, b1, a1, scale, beta, w2t, b2, a2)


def _final_kernel(h_ref, p_ref, w1t_ref, b1_ref, a1_ref, scale_ref,
                  beta_ref, w2t_ref, b2_ref, a2_ref, batch_ref, adj_ref,
                  fc1adjt_ref, stats_ref, fc1poolt_ref, fc1statst_ref,
                  fc1b_ref, fc2t_ref, fc2b_ref, xl_ref, out_ref,
                  pool_acc, adj_acc):
    i = pl.program_id(0)
    z = h_ref[...] + p_ref[0] + p_ref[1]
    h = _mlp_block(
        z, w1t_ref[...], b1_ref[...], a1_ref[0, 0],
        scale_ref[...], beta_ref[...], w2t_ref[...], b2_ref[...],
        a2_ref[0, 0])

    b_ids = batch_ref[0, 0, :]
    seg = lax.broadcasted_iota(jnp.int32, (BPAD, ROW_BLOCK), 0)
    onehot_t = (seg == jnp.reshape(b_ids, (1, ROW_BLOCK))).astype(jnp.float32)
    pool_contrib = _dot(onehot_t, h)
    adj_contrib = _dot(adj_ref[...], fc1adjt_ref[...])

    @pl.when(i == 0)
    def _():
        pool_acc[...] = pool_contrib
        adj_acc[...] = adj_contrib

    @pl.when(i != 0)
    def _():
        pool_acc[...] += pool_contrib
        adj_acc[...] += adj_contrib

    @pl.when(i == N_BLOCKS - 1)
    def _():
        xt = (_dot(pool_acc[...], fc1poolt_ref[...])
              + _dot(stats_ref[...], fc1statst_ref[...])
              + adj_acc[...] + fc1b_ref[...])
        xt = jnp.maximum(xt, 0.0)
        xl_ref[...] = xt
        logits = _dot(xt, fc2t_ref[...]) + fc2b_ref[...]
        out_ref[...] = jax.nn.sigmoid(logits)


def _final_call(h, p, w1t, b1, a1, scale, beta, w2t, b2, a2, batch3d,
                adj_pad, fc1adjt, stats_pad, fc1poolt, fc1statst, fc1b,
                fc2t, fc2b):
    fixed = lambda i: (0, 0)
    return pl.pallas_call(
        _final_kernel,
        grid=(N_BLOCKS,),
        in_specs=[
            pl.BlockSpec((ROW_BLOCK, DIM), lambda i: (i, 0)),
            pl.BlockSpec((2, ROW_BLOCK, DIM), lambda i: (0, i, 0)),
            pl.BlockSpec((DIM, DIM), fixed),
            pl.BlockSpec((1, DIM), fixed),
            pl.BlockSpec((1, 1), fixed),
            pl.BlockSpec((1, DIM), fixed),
            pl.BlockSpec((1, DIM), fixed),
            pl.BlockSpec((DIM, DIM), fixed),
            pl.BlockSpec((1, DIM), fixed),
            pl.BlockSpec((1, 1), fixed),
            pl.BlockSpec((1, 1, ROW_BLOCK), lambda i: (i, 0, 0)),
            pl.BlockSpec((BPAD, ADJ_BLOCK), lambda i: (0, i)),
            pl.BlockSpec((ADJ_BLOCK, DIM), lambda i: (i, 0)),
            pl.BlockSpec((BPAD, DIM), fixed),
            pl.BlockSpec((DIM, DIM), fixed),
            pl.BlockSpec((DIM, DIM), fixed),
            pl.BlockSpec((1, DIM), fixed),
            pl.BlockSpec((DIM, DIM), fixed),
            pl.BlockSpec((1, DIM), fixed),
        ],
        out_specs=[
            pl.BlockSpec((BPAD, DIM), fixed),
            pl.BlockSpec((BPAD, DIM), fixed),
        ],
        out_shape=[
            jax.ShapeDtypeStruct((BPAD, DIM), jnp.float32),
            jax.ShapeDtypeStruct((BPAD, DIM), jnp.float32),
        ],
        scratch_shapes=[
            pltpu.VMEM((BPAD, DIM), jnp.float32),
            pltpu.VMEM((BPAD, DIM), jnp.float32),
        ],
    )(h, p, w1t, b1, a1, scale, beta, w2t, b2, a2, batch3d, adj_pad,
      fc1adjt, stats_pad, fc1poolt, fc1statst, fc1b, fc2t, fc2b)


def kernel(x, edge_index, batch, stats, adj, W1, b1, a1, bn_gamma, bn_beta,
           W2, b2, a2, fc1_w, fc1_b, fc2_w, fc2_b):
    f32 = jnp.float32

    src = edge_index[0].astype(jnp.int32)
    dst = edge_index[1].astype(jnp.int32)
    pad_e = EPAD - N_EDGES
    src_p = jnp.concatenate([src, jnp.zeros((pad_e,), jnp.int32)])
    dummy_dst = DUMMY_ROW + (jnp.arange(pad_e, dtype=jnp.int32) % N_DUMMY)
    dst_p = jnp.concatenate([dst, dummy_dst])

    n0 = 16 * CPW0 * CHUNK
    fill_n = (CPW0 - CPW1) * CHUNK
    filler_src = jnp.zeros((16, fill_n), jnp.int32)
    filler_dst = jnp.broadcast_to(
        DUMMY_ROW + (jnp.arange(fill_n, dtype=jnp.int32) % N_DUMMY),
        (16, fill_n))

    def split_rows(flat, filler):
        rows0 = flat[:n0].reshape(16, CPW0 * CHUNK)
        rows1 = jnp.concatenate(
            [flat[n0:].reshape(16, CPW1 * CHUNK), filler], axis=1)
        return jnp.stack([rows0, rows1], axis=1).reshape(32, CPW0 * CHUNK)

    src2 = split_rows(src_p, filler_src)
    dst3 = split_rows(dst_p, filler_dst).reshape(32, CPW0, CHUNK)

    h = jnp.concatenate(
        [x.astype(f32), jnp.zeros((NPAD - N_NODES, DIM), f32)], axis=0)
    zeros_h = jnp.zeros((NPAD, DIM), f32)

    bn_scale = (bn_gamma * jax.lax.rsqrt(jnp.float32(1.0 + 1e-5))).astype(f32)

    batch_p = jnp.concatenate(
        [batch.astype(jnp.int32),
         jnp.full((NPAD - N_NODES,), BPAD - 1, jnp.int32)])
    batch3d = batch_p.reshape(N_BLOCKS, 1, ROW_BLOCK)

    adj_flat = adj.reshape(N_GRAPHS, ADJ_COLS).astype(f32)
    adj_pad = jnp.zeros((BPAD, ADJ_PAD), f32)
    adj_pad = adj_pad.at[:N_GRAPHS, :ADJ_COLS].set(adj_flat)

    stats_pad = jnp.zeros((BPAD, DIM), f32)
    stats_pad = stats_pad.at[:N_GRAPHS, :N_STATS].set(stats.astype(f32))

    fc1 = fc1_w.astype(f32)
    fc1poolt = jnp.zeros((DIM, DIM), f32)
    fc1poolt = fc1poolt.at[:, :HID].set(fc1[:, :DIM].T)
    fc1statst = jnp.zeros((DIM, DIM), f32)
    fc1statst = fc1statst.at[:N_STATS, :HID].set(fc1[:, DIM:DIM + N_STATS].T)
    fc1adjt = jnp.zeros((ADJ_PAD, DIM), f32)
    fc1adjt = fc1adjt.at[:ADJ_COLS, :HID].set(fc1[:, DIM + N_STATS:].T)
    fc1b = jnp.zeros((1, DIM), f32)
    fc1b = fc1b.at[0, :HID].set(fc1_b.astype(f32))
    fc2t = jnp.zeros((DIM, DIM), f32)
    fc2t = fc2t.at[:HID, 0].set(fc2_w.astype(f32)[0, :])
    fc2b = jnp.zeros((1, DIM), f32)
    fc2b = fc2b.at[0, 0].set(fc2_b.astype(f32)[0])

    for i in range(N_LAYERS):
        p = _sc_agg(h, zeros_h, src2, dst3)
        w1t = W1[i].T.astype(f32)
        w2t = W2[i].T.astype(f32)
        b1r = b1[i].reshape(1, DIM).astype(f32)
        b2r = b2[i].reshape(1, DIM).astype(f32)
        a1s = a1[i].reshape(1, 1).astype(f32)
        a2s = a2[i].reshape(1, 1).astype(f32)
        sc_r = bn_scale[i].reshape(1, DIM)
        be_r = bn_beta[i].reshape(1, DIM).astype(f32)
        if i < N_LAYERS - 1:
            h = _mlp_call(h, p, w1t, b1r, a1s, sc_r, be_r, w2t, b2r, a2s)
        else:
            xl, oo = _final_call(h, p, w1t, b1r, a1s, sc_r, be_r, w2t,
                                 b2r, a2s, batch3d, adj_pad, fc1adjt,
                                 stats_pad, fc1poolt, fc1statst, fc1b,
                                 fc2t, fc2b)

    out = oo[:N_GRAPHS, :1]
    x_l = xl[:N_GRAPHS, :HID]
    return (out, x_l)


# speedup vs baseline: 1.2462x; 1.0881x over previous
import functools

import jax
import jax.numpy as jnp
from jax import lax
from jax.experimental import pallas as pl
from jax.experimental.pallas import tpu as pltpu
from jax.experimental.pallas import tpu_sc as plsc

N_NODES = 10000
N_EDGES = 320000
DIM = 128
N_GRAPHS = 200
N_LAYERS = 3
N_STATS = 7
N_MOTIF = 50

NPAD = 10112
DUMMY_ROW = 10000
N_DUMMY = NPAD - DUMMY_ROW
CHUNK = 128
EPAD = 327680
N_CHUNKS = EPAD // CHUNK
CPW0 = 128
CPW1 = 32
ROWS_PER_TILE = NPAD // 16

BPAD = 256
ROW_BLOCK = NPAD // 8
N_BLOCKS = 8
ADJ_COLS = N_MOTIF * N_MOTIF
ADJ_PAD = 3072
ADJ_BLOCK = ADJ_PAD // N_BLOCKS
HID = DIM // 2


def _sc_agg_body(h_hbm, zeros_hbm, src_hbm, dst_hbm, out,
                 src_v, ring, rows0, rows1, acc_sh,
                 sg0, sg1, ss0, ss1, sr0, sr1):
    cid = lax.axis_index("c")
    sid = lax.axis_index("s")
    wid = sid * 2 + cid
    row0 = sid * ROWS_PER_TILE
    rows = (rows0, rows1)
    sg = (sg0, sg1)
    ss = (ss0, ss1)
    sr = (sr0, sr1)
    cpw = jnp.where(cid == 0, CPW1, CPW0)

    pltpu.sync_copy(zeros_hbm.at[pl.ds(row0, ROWS_PER_TILE)],
                    acc_sh.at[pl.ds(row0, ROWS_PER_TILE)])

    pltpu.sync_copy(src_hbm.at[wid], src_v)
    plsc.subcore_barrier()

    def src_ix(j):
        return src_v.at[pl.ds(pl.multiple_of(j * CHUNK, 8), CHUNK)]

    pltpu.async_copy(dst_hbm.at[wid, 0], ring.at[0], sr0)
    pltpu.async_copy(h_hbm.at[src_ix(0)], rows0, sg0)

    @pl.loop(0, cpw, step=2)
    def _(j0):
        for b in range(2):
            j = j0 + b
            bg = 1 - b

            @pl.when(j >= 1)
            def _():
                pltpu.make_async_copy(
                    rows[bg], acc_sh.at[ring.at[bg]], ss[bg]).wait()

            @pl.when(j + 1 < cpw)
            def _():
                pltpu.async_copy(dst_hbm.at[wid, j + 1], ring.at[bg],
                                 sr[bg])
                pltpu.async_copy(h_hbm.at[src_ix(j + 1)], rows[bg], sg[bg])

            pltpu.make_async_copy(h_hbm.at[src_ix(j)], rows[b],
                                  sg[b]).wait()
            pltpu.make_async_copy(dst_hbm.at[wid, j], ring.at[b],
                                  sr[b]).wait()
            pltpu.async_copy(rows[b], acc_sh.at[ring.at[b]], ss[b],
                             add=True)

    pltpu.make_async_copy(rows1, acc_sh.at[ring.at[1]], ss1).wait()

    plsc.subcore_barrier()
    pltpu.sync_copy(acc_sh.at[pl.ds(row0, ROWS_PER_TILE)],
                    out.at[cid, pl.ds(row0, ROWS_PER_TILE)])


@functools.lru_cache(maxsize=1)
def _make_sc_agg():
    return pl.kernel(
        _sc_agg_body,
        out_type=jax.ShapeDtypeStruct((2, NPAD, DIM), jnp.float32),
        mesh=plsc.VectorSubcoreMesh(core_axis_name="c",
                                    subcore_axis_name="s"),
        scratch_types=[
            pltpu.VMEM((CPW0 * CHUNK,), jnp.int32),
            pltpu.VMEM((2, CHUNK), jnp.int32),
            pltpu.VMEM((CHUNK, DIM), jnp.float32),
            pltpu.VMEM((CHUNK, DIM), jnp.float32),
            pltpu.VMEM_SHARED((NPAD, DIM), jnp.float32),
            pltpu.SemaphoreType.DMA,
            pltpu.SemaphoreType.DMA,
            pltpu.SemaphoreType.DMA,
            pltpu.SemaphoreType.DMA,
            pltpu.SemaphoreType.DMA,
            pltpu.SemaphoreType.DMA,
        ],
    )


def _sc_agg(h, zeros_h, src2, dst3):
    return _make_sc_agg()(h, zeros_h, src2, dst3)


def _dot(a, b):
    return lax.dot_general(a, b, (((1,), (0,)), ((), ())),
                           precision=lax.Precision.HIGHEST,
                           preferred_element_type=jnp.float32)


def _mlp_block(z, w1t, b1, a1, scale, beta, w2t, b2, a2):
    z = _dot(z, w1t) + b1
    z = jnp.where(z >= 0, z, a1 * z)
    z = z * scale + beta
    z = _dot(z, w2t) + b2
    z = jnp.where(z >= 0, z, a2 * z)
    return jnp.maximum(z, 0.0)


def _mlp_kernel(h_ref, p_ref, w1t_ref, b1_ref, a1_ref, scale_ref, beta_ref,
                w2t_ref, b2_ref, a2_ref, out_ref):
    z = h_ref[...] + p_ref[0] + p_ref[1]
    out_ref[...] = _mlp_block(
        z, w1t_ref[...], b1_ref[...], a1_ref[0, 0],
        scale_ref[...], beta_ref[...], w2t_ref[...], b2_ref[...],
        a2_ref[0, 0])


def _mlp_call(h, p, w1t, b1, a1, scale, beta, w2t, b2, a2):
    blk = lambda i: (i, 0)
    fixed = lambda i: (0, 0)
    return pl.pallas_call(
        _mlp_kernel,
        grid=(N_BLOCKS,),
        in_specs=[
            pl.BlockSpec((ROW_BLOCK, DIM), blk),
            pl.BlockSpec((2, ROW_BLOCK, DIM), lambda i: (0, i, 0)),
            pl.BlockSpec((DIM, DIM), fixed),
            pl.BlockSpec((1, DIM), fixed),
            pl.BlockSpec((1, 1), fixed),
            pl.BlockSpec((1, DIM), fixed),
            pl.BlockSpec((1, DIM), fixed),
            pl.BlockSpec((DIM, DIM), fixed),
            pl.BlockSpec((1, DIM), fixed),
            pl.BlockSpec((1, 1), fixed),
        ],
        out_specs=pl.BlockSpec((ROW_BLOCK, DIM), blk),
        out_shape=jax.ShapeDtypeStruct((NPAD, DIM), jnp.float32),
    )(h, p, w1t, b1, a1, scale, beta, w2t, b2, a2)


def _final_kernel(h_ref, p_ref, w1t_ref, b1_ref, a1_ref, scale_ref,
                  beta_ref, w2t_ref, b2_ref, a2_ref, batch_ref, adj_ref,
                  fc1adjt_ref, stats_ref, fc1poolt_ref, fc1statst_ref,
                  fc1b_ref, fc2t_ref, fc2b_ref, xl_ref, out_ref,
                  pool_acc, adj_acc):
    i = pl.program_id(0)
    z = h_ref[...] + p_ref[0] + p_ref[1]
    h = _mlp_block(
        z, w1t_ref[...], b1_ref[...], a1_ref[0, 0],
        scale_ref[...], beta_ref[...], w2t_ref[...], b2_ref[...],
        a2_ref[0, 0])

    b_ids = batch_ref[0, 0, :]
    seg = lax.broadcasted_iota(jnp.int32, (BPAD, ROW_BLOCK), 0)
    onehot_t = (seg == jnp.reshape(b_ids, (1, ROW_BLOCK))).astype(jnp.float32)
    pool_contrib = _dot(onehot_t, h)
    adj_contrib = _dot(adj_ref[...], fc1adjt_ref[...])

    @pl.when(i == 0)
    def _():
        pool_acc[...] = pool_contrib
        adj_acc[...] = adj_contrib

    @pl.when(i != 0)
    def _():
        pool_acc[...] += pool_contrib
        adj_acc[...] += adj_contrib

    @pl.when(i == N_BLOCKS - 1)
    def _():
        xt = (_dot(pool_acc[...], fc1poolt_ref[...])
              + _dot(stats_ref[...], fc1statst_ref[...])
              + adj_acc[...] + fc1b_ref[...])
        xt = jnp.maximum(xt, 0.0)
        xl_ref[...] = xt
        logits = _dot(xt, fc2t_ref[...]) + fc2b_ref[...]
        out_ref[...] = jax.nn.sigmoid(logits)


def _final_call(h, p, w1t, b1, a1, scale, beta, w2t, b2, a2, batch3d,
                adj_pad, fc1adjt, stats_pad, fc1poolt, fc1statst, fc1b,
                fc2t, fc2b):
    fixed = lambda i: (0, 0)
    return pl.pallas_call(
        _final_kernel,
        grid=(N_BLOCKS,),
        in_specs=[
            pl.BlockSpec((ROW_BLOCK, DIM), lambda i: (i, 0)),
            pl.BlockSpec((2, ROW_BLOCK, DIM), lambda i: (0, i, 0)),
            pl.BlockSpec((DIM, DIM), fixed),
            pl.BlockSpec((1, DIM), fixed),
            pl.BlockSpec((1, 1), fixed),
            pl.BlockSpec((1, DIM), fixed),
            pl.BlockSpec((1, DIM), fixed),
            pl.BlockSpec((DIM, DIM), fixed),
            pl.BlockSpec((1, DIM), fixed),
            pl.BlockSpec((1, 1), fixed),
            pl.BlockSpec((1, 1, ROW_BLOCK), lambda i: (i, 0, 0)),
            pl.BlockSpec((BPAD, ADJ_BLOCK), lambda i: (0, i)),
            pl.BlockSpec((ADJ_BLOCK, DIM), lambda i: (i, 0)),
            pl.BlockSpec((BPAD, DIM), fixed),
            pl.BlockSpec((DIM, DIM), fixed),
            pl.BlockSpec((DIM, DIM), fixed),
            pl.BlockSpec((1, DIM), fixed),
            pl.BlockSpec((DIM, DIM), fixed),
            pl.BlockSpec((1, DIM), fixed),
        ],
        out_specs=[
            pl.BlockSpec((BPAD, DIM), fixed),
            pl.BlockSpec((BPAD, DIM), fixed),
        ],
        out_shape=[
            jax.ShapeDtypeStruct((BPAD, DIM), jnp.float32),
            jax.ShapeDtypeStruct((BPAD, DIM), jnp.float32),
        ],
        scratch_shapes=[
            pltpu.VMEM((BPAD, DIM), jnp.float32),
            pltpu.VMEM((BPAD, DIM), jnp.float32),
        ],
    )(h, p, w1t, b1, a1, scale, beta, w2t, b2, a2, batch3d, adj_pad,
      fc1adjt, stats_pad, fc1poolt, fc1statst, fc1b, fc2t, fc2b)


def kernel(x, edge_index, batch, stats, adj, W1, b1, a1, bn_gamma, bn_beta,
           W2, b2, a2, fc1_w, fc1_b, fc2_w, fc2_b):
    f32 = jnp.float32

    src = edge_index[0].astype(jnp.int32)
    dst = edge_index[1].astype(jnp.int32)
    pad_e = EPAD - N_EDGES
    src_p = jnp.concatenate([src, jnp.zeros((pad_e,), jnp.int32)])
    dummy_dst = DUMMY_ROW + (jnp.arange(pad_e, dtype=jnp.int32) % N_DUMMY)
    dst_p = jnp.concatenate([dst, dummy_dst])

    n0 = 16 * CPW0 * CHUNK
    filler_src = jnp.zeros((16, (CPW0 - CPW1) * CHUNK), jnp.int32)
    filler_dst = jnp.broadcast_to(
        DUMMY_ROW + (jnp.arange((CPW0 - CPW1) * CHUNK, dtype=jnp.int32)
                     % N_DUMMY),
        (16, (CPW0 - CPW1) * CHUNK))

    def split_rows(flat, filler):
        rows0 = flat[:n0].reshape(16, CPW0 * CHUNK)
        rows1 = jnp.concatenate(
            [flat[n0:].reshape(16, CPW1 * CHUNK), filler], axis=1)
        return jnp.stack([rows1, rows0], axis=1).reshape(32, CPW0 * CHUNK)

    src2 = split_rows(src_p, filler_src)
    dst3 = split_rows(dst_p, filler_dst).reshape(32, CPW0, CHUNK)

    h = jnp.concatenate(
        [x.astype(f32), jnp.zeros((NPAD - N_NODES, DIM), f32)], axis=0)
    zeros_h = jnp.zeros((NPAD, DIM), f32)

    bn_scale = (bn_gamma * jax.lax.rsqrt(jnp.float32(1.0 + 1e-5))).astype(f32)

    batch_p = jnp.concatenate(
        [batch.astype(jnp.int32),
         jnp.full((NPAD - N_NODES,), BPAD - 1, jnp.int32)])
    batch3d = batch_p.reshape(N_BLOCKS, 1, ROW_BLOCK)

    adj_flat = adj.reshape(N_GRAPHS, ADJ_COLS).astype(f32)
    adj_pad = jnp.zeros((BPAD, ADJ_PAD), f32)
    adj_pad = adj_pad.at[:N_GRAPHS, :ADJ_COLS].set(adj_flat)

    stats_pad = jnp.zeros((BPAD, DIM), f32)
    stats_pad = stats_pad.at[:N_GRAPHS, :N_STATS].set(stats.astype(f32))

    fc1 = fc1_w.astype(f32)
    fc1poolt = jnp.zeros((DIM, DIM), f32)
    fc1poolt = fc1poolt.at[:, :HID].set(fc1[:, :DIM].T)
    fc1statst = jnp.zeros((DIM, DIM), f32)
    fc1statst = fc1statst.at[:N_STATS, :HID].set(fc1[:, DIM:DIM + N_STATS].T)
    fc1adjt = jnp.zeros((ADJ_PAD, DIM), f32)
    fc1adjt = fc1adjt.at[:ADJ_COLS, :HID].set(fc1[:, DIM + N_STATS:].T)
    fc1b = jnp.zeros((1, DIM), f32)
    fc1b = fc1b.at[0, :HID].set(fc1_b.astype(f32))
    fc2t = jnp.zeros((DIM, DIM), f32)
    fc2t = fc2t.at[:HID, 0].set(fc2_w.astype(f32)[0, :])
    fc2b = jnp.zeros((1, DIM), f32)
    fc2b = fc2b.at[0, 0].set(fc2_b.astype(f32)[0])

    for i in range(N_LAYERS):
        p = _sc_agg(h, zeros_h, src2, dst3)
        w1t = W1[i].T.astype(f32)
        w2t = W2[i].T.astype(f32)
        b1r = b1[i].reshape(1, DIM).astype(f32)
        b2r = b2[i].reshape(1, DIM).astype(f32)
        a1s = a1[i].reshape(1, 1).astype(f32)
        a2s = a2[i].reshape(1, 1).astype(f32)
        sc_r = bn_scale[i].reshape(1, DIM)
        be_r = bn_beta[i].reshape(1, DIM).astype(f32)
        if i < N_LAYERS - 1:
            h = _mlp_call(h, p, w1t, b1r, a1s, sc_r, be_r, w2t, b2r, a2s)
        else:
            xl, oo = _final_call(h, p, w1t, b1r, a1s, sc_r, be_r, w2t,
                                 b2r, a2s, batch3d, adj_pad, fc1adjt,
                                 stats_pad, fc1poolt, fc1statst, fc1b,
                                 fc2t, fc2b)

    out = oo[:N_GRAPHS, :1]
    x_l = xl[:N_GRAPHS, :HID]
    return (out, x_l)


# speedup vs baseline: 1.3260x; 1.0640x over previous
import functools

import jax
import jax.numpy as jnp
from jax import lax
from jax.experimental import pallas as pl
from jax.experimental.pallas import tpu as pltpu
from jax.experimental.pallas import tpu_sc as plsc

N_NODES = 10000
N_EDGES = 320000
DIM = 128
N_GRAPHS = 200
N_LAYERS = 3
N_STATS = 7
N_MOTIF = 50

NPAD = 10112
DUMMY_ROW = 10000
N_DUMMY = NPAD - DUMMY_ROW
CHUNK = 128
EPAD = 327680
N_CHUNKS = EPAD // CHUNK
CPW0 = 128
CPW1 = 32
ROWS_PER_TILE = NPAD // 16

BPAD = 256
ROW_BLOCK = NPAD // 8
N_BLOCKS = 8
ADJ_COLS = N_MOTIF * N_MOTIF
ADJ_PAD = 3072
ADJ_BLOCK = ADJ_PAD // N_BLOCKS
HID = DIM // 2


def _sc_agg_body(h_hbm, zeros_hbm, src_hbm, dst_hbm, out,
                 src_v, ring, rows0, rows1, acc_sh,
                 sg0, sg1, ss0, ss1, sr0, sr1):
    cid = lax.axis_index("c")
    sid = lax.axis_index("s")
    wid = sid * 2 + cid
    row0 = sid * ROWS_PER_TILE
    rows = (rows0, rows1)
    sg = (sg0, sg1)
    ss = (ss0, ss1)
    sr = (sr0, sr1)
    cpw = jnp.where(cid == 0, CPW0, CPW1)

    pltpu.sync_copy(zeros_hbm.at[pl.ds(row0, ROWS_PER_TILE)],
                    acc_sh.at[pl.ds(row0, ROWS_PER_TILE)])

    pltpu.sync_copy(src_hbm.at[wid], src_v)
    plsc.subcore_barrier()

    def src_ix(j):
        return src_v.at[pl.ds(pl.multiple_of(j * CHUNK, 8), CHUNK)]

    pltpu.async_copy(dst_hbm.at[wid, 0], ring.at[0], sr0)
    pltpu.async_copy(h_hbm.at[src_ix(0)], rows0, sg0)

    @pl.loop(0, cpw, step=2)
    def _(j0):
        for b in range(2):
            j = j0 + b
            bg = 1 - b

            @pl.when(j >= 1)
            def _():
                pltpu.make_async_copy(
                    rows[bg], acc_sh.at[ring.at[bg]], ss[bg]).wait()

            @pl.when(j + 1 < cpw)
            def _():
                pltpu.async_copy(dst_hbm.at[wid, j + 1], ring.at[bg],
                                 sr[bg])
                pltpu.async_copy(h_hbm.at[src_ix(j + 1)], rows[bg], sg[bg])

            pltpu.make_async_copy(h_hbm.at[src_ix(j)], rows[b],
                                  sg[b]).wait()
            pltpu.make_async_copy(dst_hbm.at[wid, j], ring.at[b],
                                  sr[b]).wait()
            pltpu.async_copy(rows[b], acc_sh.at[ring.at[b]], ss[b],
                             add=True)

    pltpu.make_async_copy(rows1, acc_sh.at[ring.at[1]], ss1).wait()

    plsc.subcore_barrier()
    pltpu.sync_copy(acc_sh.at[pl.ds(row0, ROWS_PER_TILE)],
                    out.at[cid, pl.ds(row0, ROWS_PER_TILE)])


@functools.lru_cache(maxsize=1)
def _make_sc_agg():
    return pl.kernel(
        _sc_agg_body,
        out_type=jax.ShapeDtypeStruct((2, NPAD, DIM), jnp.float32),
        mesh=plsc.VectorSubcoreMesh(core_axis_name="c",
                                    subcore_axis_name="s"),
        scratch_types=[
            pltpu.VMEM((CPW0 * CHUNK,), jnp.int32),
            pltpu.VMEM((2, CHUNK), jnp.int32),
            pltpu.VMEM((CHUNK, DIM), jnp.float32),
            pltpu.VMEM((CHUNK, DIM), jnp.float32),
            pltpu.VMEM_SHARED((NPAD, DIM), jnp.float32),
            pltpu.SemaphoreType.DMA,
            pltpu.SemaphoreType.DMA,
            pltpu.SemaphoreType.DMA,
            pltpu.SemaphoreType.DMA,
            pltpu.SemaphoreType.DMA,
            pltpu.SemaphoreType.DMA,
        ],
    )


def _sc_agg(h, zeros_h, src2, dst3):
    return _make_sc_agg()(h, zeros_h, src2, dst3)


def _dot(a, b):
    return lax.dot_general(a, b, (((1,), (0,)), ((), ())),
                           preferred_element_type=jnp.float32)


def _mlp_block(z, w1t, b1, a1, scale, beta, w2t, b2, a2):
    z = _dot(z, w1t) + b1
    z = jnp.where(z >= 0, z, a1 * z)
    z = z * scale + beta
    z = _dot(z, w2t) + b2
    z = jnp.where(z >= 0, z, a2 * z)
    return jnp.maximum(z, 0.0)


def _mlp_kernel(h_ref, p_ref, w1t_ref, b1_ref, a1_ref, scale_ref, beta_ref,
                w2t_ref, b2_ref, a2_ref, out_ref):
    z = h_ref[...] + p_ref[0] + p_ref[1]
    out_ref[...] = _mlp_block(
        z, w1t_ref[...], b1_ref[...], a1_ref[0, 0],
        scale_ref[...], beta_ref[...], w2t_ref[...], b2_ref[...],
        a2_ref[0, 0])


def _mlp_call(h, p, w1t, b1, a1, scale, beta, w2t, b2, a2):
    blk = lambda i: (i, 0)
    fixed = lambda i: (0, 0)
    return pl.pallas_call(
        _mlp_kernel,
        grid=(N_BLOCKS,),
        in_specs=[
            pl.BlockSpec((ROW_BLOCK, DIM), blk),
            pl.BlockSpec((2, ROW_BLOCK, DIM), lambda i: (0, i, 0)),
            pl.BlockSpec((DIM, DIM), fixed),
            pl.BlockSpec((1, DIM), fixed),
            pl.BlockSpec((1, 1), fixed),
            pl.BlockSpec((1, DIM), fixed),
            pl.BlockSpec((1, DIM), fixed),
            pl.BlockSpec((DIM, DIM), fixed),
            pl.BlockSpec((1, DIM), fixed),
            pl.BlockSpec((1, 1), fixed),
        ],
        out_specs=pl.BlockSpec((ROW_BLOCK, DIM), blk),
        out_shape=jax.ShapeDtypeStruct((NPAD, DIM), jnp.float32),
    )(h, p, w1t, b1, a1, scale, beta, w2t, b2, a2)


def _final_kernel(h_ref, p_ref, w1t_ref, b1_ref, a1_ref, scale_ref,
                  beta_ref, w2t_ref, b2_ref, a2_ref, batch_ref, adj_ref,
                  fc1adjt_ref, stats_ref, fc1poolt_ref, fc1statst_ref,
                  fc1b_ref, fc2t_ref, fc2b_ref, xl_ref, out_ref,
                  pool_acc, adj_acc):
    i = pl.program_id(0)
    z = h_ref[...] + p_ref[0] + p_ref[1]
    h = _mlp_block(
        z, w1t_ref[...], b1_ref[...], a1_ref[0, 0],
        scale_ref[...], beta_ref[...], w2t_ref[...], b2_ref[...],
        a2_ref[0, 0])

    b_ids = batch_ref[0, 0, :]
    seg = lax.broadcasted_iota(jnp.int32, (BPAD, ROW_BLOCK), 0)
    onehot_t = (seg == jnp.reshape(b_ids, (1, ROW_BLOCK))).astype(jnp.float32)
    pool_contrib = _dot(onehot_t, h)
    adj_contrib = _dot(adj_ref[...], fc1adjt_ref[...])

    @pl.when(i == 0)
    def _():
        pool_acc[...] = pool_contrib
        adj_acc[...] = adj_contrib

    @pl.when(i != 0)
    def _():
        pool_acc[...] += pool_contrib
        adj_acc[...] += adj_contrib

    @pl.when(i == N_BLOCKS - 1)
    def _():
        xt = (_dot(pool_acc[...], fc1poolt_ref[...])
              + _dot(stats_ref[...], fc1statst_ref[...])
              + adj_acc[...] + fc1b_ref[...])
        xt = jnp.maximum(xt, 0.0)
        xl_ref[...] = xt
        logits = _dot(xt, fc2t_ref[...]) + fc2b_ref[...]
        out_ref[...] = jax.nn.sigmoid(logits)


def _final_call(h, p, w1t, b1, a1, scale, beta, w2t, b2, a2, batch3d,
                adj_pad, fc1adjt, stats_pad, fc1poolt, fc1statst, fc1b,
                fc2t, fc2b):
    fixed = lambda i: (0, 0)
    return pl.pallas_call(
        _final_kernel,
        grid=(N_BLOCKS,),
        in_specs=[
            pl.BlockSpec((ROW_BLOCK, DIM), lambda i: (i, 0)),
            pl.BlockSpec((2, ROW_BLOCK, DIM), lambda i: (0, i, 0)),
            pl.BlockSpec((DIM, DIM), fixed),
            pl.BlockSpec((1, DIM), fixed),
            pl.BlockSpec((1, 1), fixed),
            pl.BlockSpec((1, DIM), fixed),
            pl.BlockSpec((1, DIM), fixed),
            pl.BlockSpec((DIM, DIM), fixed),
            pl.BlockSpec((1, DIM), fixed),
            pl.BlockSpec((1, 1), fixed),
            pl.BlockSpec((1, 1, ROW_BLOCK), lambda i: (i, 0, 0)),
            pl.BlockSpec((BPAD, ADJ_BLOCK), lambda i: (0, i)),
            pl.BlockSpec((ADJ_BLOCK, DIM), lambda i: (i, 0)),
            pl.BlockSpec((BPAD, DIM), fixed),
            pl.BlockSpec((DIM, DIM), fixed),
            pl.BlockSpec((DIM, DIM), fixed),
            pl.BlockSpec((1, DIM), fixed),
            pl.BlockSpec((DIM, DIM), fixed),
            pl.BlockSpec((1, DIM), fixed),
        ],
        out_specs=[
            pl.BlockSpec((BPAD, DIM), fixed),
            pl.BlockSpec((BPAD, DIM), fixed),
        ],
        out_shape=[
            jax.ShapeDtypeStruct((BPAD, DIM), jnp.float32),
            jax.ShapeDtypeStruct((BPAD, DIM), jnp.float32),
        ],
        scratch_shapes=[
            pltpu.VMEM((BPAD, DIM), jnp.float32),
            pltpu.VMEM((BPAD, DIM), jnp.float32),
        ],
    )(h, p, w1t, b1, a1, scale, beta, w2t, b2, a2, batch3d, adj_pad,
      fc1adjt, stats_pad, fc1poolt, fc1statst, fc1b, fc2t, fc2b)


def kernel(x, edge_index, batch, stats, adj, W1, b1, a1, bn_gamma, bn_beta,
           W2, b2, a2, fc1_w, fc1_b, fc2_w, fc2_b):
    f32 = jnp.float32

    src = edge_index[0].astype(jnp.int32)
    dst = edge_index[1].astype(jnp.int32)
    pad_e = EPAD - N_EDGES
    src_p = jnp.concatenate([src, jnp.zeros((pad_e,), jnp.int32)])
    dummy_dst = DUMMY_ROW + (jnp.arange(pad_e, dtype=jnp.int32) % N_DUMMY)
    dst_p = jnp.concatenate([dst, dummy_dst])

    n0 = 16 * CPW0 * CHUNK
    filler_src = jnp.zeros((16, (CPW0 - CPW1) * CHUNK), jnp.int32)
    filler_dst = jnp.broadcast_to(
        DUMMY_ROW + (jnp.arange((CPW0 - CPW1) * CHUNK, dtype=jnp.int32)
                     % N_DUMMY),
        (16, (CPW0 - CPW1) * CHUNK))

    def split_rows(flat, filler):
        rows0 = flat[:n0].reshape(16, CPW0 * CHUNK)
        rows1 = jnp.concatenate(
            [flat[n0:].reshape(16, CPW1 * CHUNK), filler], axis=1)
        return jnp.stack([rows0, rows1], axis=1).reshape(32, CPW0 * CHUNK)

    src2 = split_rows(src_p, filler_src)
    dst3 = split_rows(dst_p, filler_dst).reshape(32, CPW0, CHUNK)

    h = jnp.concatenate(
        [x.astype(f32), jnp.zeros((NPAD - N_NODES, DIM), f32)], axis=0)
    zeros_h = jnp.zeros((NPAD, DIM), f32)

    bn_scale = (bn_gamma * jax.lax.rsqrt(jnp.float32(1.0 + 1e-5))).astype(f32)

    batch_p = jnp.concatenate(
        [batch.astype(jnp.int32),
         jnp.full((NPAD - N_NODES,), BPAD - 1, jnp.int32)])
    batch3d = batch_p.reshape(N_BLOCKS, 1, ROW_BLOCK)

    adj_flat = adj.reshape(N_GRAPHS, ADJ_COLS).astype(f32)
    adj_pad = jnp.zeros((BPAD, ADJ_PAD), f32)
    adj_pad = adj_pad.at[:N_GRAPHS, :ADJ_COLS].set(adj_flat)

    stats_pad = jnp.zeros((BPAD, DIM), f32)
    stats_pad = stats_pad.at[:N_GRAPHS, :N_STATS].set(stats.astype(f32))

    fc1 = fc1_w.astype(f32)
    fc1poolt = jnp.zeros((DIM, DIM), f32)
    fc1poolt = fc1poolt.at[:, :HID].set(fc1[:, :DIM].T)
    fc1statst = jnp.zeros((DIM, DIM), f32)
    fc1statst = fc1statst.at[:N_STATS, :HID].set(fc1[:, DIM:DIM + N_STATS].T)
    fc1adjt = jnp.zeros((ADJ_PAD, DIM), f32)
    fc1adjt = fc1adjt.at[:ADJ_COLS, :HID].set(fc1[:, DIM + N_STATS:].T)
    fc1b = jnp.zeros((1, DIM), f32)
    fc1b = fc1b.at[0, :HID].set(fc1_b.astype(f32))
    fc2t = jnp.zeros((DIM, DIM), f32)
    fc2t = fc2t.at[:HID, 0].set(fc2_w.astype(f32)[0, :])
    fc2b = jnp.zeros((1, DIM), f32)
    fc2b = fc2b.at[0, 0].set(fc2_b.astype(f32)[0])

    for i in range(N_LAYERS):
        p = _sc_agg(h, zeros_h, src2, dst3)
        w1t = W1[i].T.astype(f32)
        w2t = W2[i].T.astype(f32)
        b1r = b1[i].reshape(1, DIM).astype(f32)
        b2r = b2[i].reshape(1, DIM).astype(f32)
        a1s = a1[i].reshape(1, 1).astype(f32)
        a2s = a2[i].reshape(1, 1).astype(f32)
        sc_r = bn_scale[i].reshape(1, DIM)
        be_r = bn_beta[i].reshape(1, DIM).astype(f32)
        if i < N_LAYERS - 1:
            h = _mlp_call(h, p, w1t, b1r, a1s, sc_r, be_r, w2t, b2r, a2s)
        else:
            xl, oo = _final_call(h, p, w1t, b1r, a1s, sc_r, be_r, w2t,
                                 b2r, a2s, batch3d, adj_pad, fc1adjt,
                                 stats_pad, fc1poolt, fc1statst, fc1b,
                                 fc2t, fc2b)

    out = oo[:N_GRAPHS, :1]
    x_l = xl[:N_GRAPHS, :HID]
    return (out, x_l)
